# baseline (speedup 1.0000x reference)
import functools

import jax
import jax.numpy as jnp
from jax import lax
from jax.experimental import pallas as pl
from jax.experimental.pallas import tpu as pltpu
from jax.experimental.pallas import tpu_sc as plsc

BATCH = 16384
DIM = 32
L = 16
NC, NS = 2, 16
NW = NC * NS
BPW = BATCH // NW
CHUNK = 128
NCH = BPW // CHUNK

_mesh = plsc.VectorSubcoreMesh(core_axis_name="c", subcore_axis_name="s")


@functools.partial(
    pl.kernel,
    out_type=jax.ShapeDtypeStruct((BATCH,), jnp.float32),
    mesh=_mesh,
    compiler_params=pltpu.CompilerParams(needs_layout_passes=False,
                                         use_tc_tiling_on_sc=False),
    scratch_types=[
        pltpu.VMEM((NCH, CHUNK), jnp.int32),
        pltpu.VMEM((NCH, CHUNK), jnp.int32),
        pltpu.VMEM((BPW, DIM), jnp.float32),
        pltpu.VMEM((BPW, DIM), jnp.float32),
        pltpu.VMEM((BPW,), jnp.float32),
        pltpu.VMEM((BPW,), jnp.float32),
        pltpu.VMEM((BPW,), jnp.float32),
        pltpu.SemaphoreType.DMA,
    ],
)
def _mf_bpr(uid, iid, uemb, iemb, ubias, ibias, out,
            uidx, iidx, urows, irows, ub, ib, out_v, sem):
    wid = lax.axis_index("s") * NC + lax.axis_index("c")
    base = wid * BPW

    for j in range(NCH):
        pltpu.sync_copy(uid.at[pl.ds(base + j * CHUNK, CHUNK)], uidx.at[j])
        pltpu.sync_copy(iid.at[pl.ds(base + j * CHUNK, CHUNK)], iidx.at[j])

    for j in range(NCH):
        for k in range(CHUNK // L):
            s = pl.ds(k * L, L)
            uidx[j, s] = uidx[j, s] - 1
            iidx[j, s] = iidx[j, s] - 1

    copies = []
    for j in range(NCH):
        rs = pl.ds(j * CHUNK, CHUNK)
        copies.append(pltpu.async_copy(uemb.at[uidx.at[j]], urows.at[rs], sem))
        copies.append(pltpu.async_copy(iemb.at[iidx.at[j]], irows.at[rs], sem))
        copies.append(pltpu.async_copy(ubias.at[uidx.at[j]], ub.at[rs], sem))
        copies.append(pltpu.async_copy(ibias.at[iidx.at[j]], ib.at[rs], sem))
    for c in copies:
        c.wait()

    lanes = jnp.arange(L, dtype=jnp.int32)

    def body(t, carry):
        b0 = t * L
        acc = ub[pl.ds(b0, L)] + ib[pl.ds(b0, L)]
        for q in range(L):
            b = b0 + q
            p = (urows[b, pl.ds(0, L)] * irows[b, pl.ds(0, L)]
                 + urows[b, pl.ds(L, L)] * irows[b, pl.ds(L, L)])
            acc = acc + jnp.where(lanes == q, jnp.sum(p), 0.0)
        out_v[pl.ds(b0, L)] = acc
        return carry

    lax.fori_loop(0, BPW // L, body, 0)

    pltpu.sync_copy(out_v, out.at[pl.ds(base, BPW)])


def kernel(user_id, item_id, user_embedding, item_embedding, user_bias, item_bias):
    return _mf_bpr(user_id, item_id, user_embedding, item_embedding,
                   user_bias.reshape(-1), item_bias.reshape(-1))
